# Initial kernel scaffold; baseline (speedup 1.0000x reference)
import jax
import jax.numpy as jnp
from jax.experimental import pallas as pl


def kernel(embedding, memory_bank):
    raise NotImplementedError("write your pallas kernel here")


# speedup vs baseline: 2.5136x; 2.5136x over previous
import functools

import jax
import jax.numpy as jnp
from jax.experimental import pallas as pl
from jax.experimental.pallas import tpu as pltpu

_K = 9
_BIG_I = 2**30
_INF = float("inf")


def _body(emb_ref, bank_ref, outv_ref, outi_ref, score_ref, best_ref, *, C, nM, nQ):
    q = pl.program_id(0)
    m = pl.program_id(1)

    x = emb_ref[...]
    y = bank_ref[...]
    x2 = jnp.sum(x * x, axis=1, keepdims=True)
    y2 = jnp.sum(y * y, axis=1)[None, :]
    xy = jax.lax.dot_general(x, y, (((1,), (1,)), ((), ())),
                             preferred_element_type=jnp.float32)
    d2 = x2 + y2 - 2.0 * xy

    ids = jax.lax.broadcasted_iota(jnp.int32, d2.shape, 1) + m * C

    @pl.when(m == 0)
    def _init():
        outv_ref[...] = jnp.full(outv_ref.shape, _INF, jnp.float32)
        outi_ref[...] = jnp.zeros(outi_ref.shape, jnp.int32)

    cv = outv_ref[...]
    ci = outi_ref[...]

    newv, newi = [], []
    for _ in range(_K):
        m1 = jnp.min(d2, axis=1, keepdims=True)
        m2 = jnp.min(cv, axis=1, keepdims=True)
        v = jnp.minimum(m1, m2)
        c1 = jnp.min(jnp.where(d2 == v, ids, _BIG_I), axis=1, keepdims=True)
        c2 = jnp.min(jnp.where(cv == v, ci, _BIG_I), axis=1, keepdims=True)
        i = jnp.minimum(c1, c2)
        newv.append(v)
        newi.append(i)
        d2 = jnp.where(ids == i, _INF, d2)
        cv = jnp.where(ci == i, _INF, cv)

    vfin = jnp.concatenate(newv, axis=1)
    ifin = jnp.concatenate(newi, axis=1)
    outv_ref[...] = vfin
    outi_ref[...] = ifin

    @pl.when(m == nM - 1)
    def _finalize_tile():
        col0 = vfin[:, 0:1]
        tmax = jnp.max(col0, axis=0, keepdims=True)
        riota = jax.lax.broadcasted_iota(jnp.int32, col0.shape, 0)
        ridx = jnp.min(jnp.where(col0 == tmax, riota, _BIG_I),
                       axis=0, keepdims=True)
        row9 = jnp.sum(jnp.where(riota == ridx, vfin, 0.0),
                       axis=0, keepdims=True)

        outv_ref[...] = jnp.sqrt(jnp.maximum(vfin, 1e-12))

        @pl.when(q == 0)
        def _():
            best_ref[0:1, 0:_K] = row9
            best_ref[1:2, 0:1] = tmax

        @pl.when(q > 0)
        def _():
            prev = best_ref[1:2, 0:1]
            take = tmax > prev
            best_ref[0:1, 0:_K] = jnp.where(take, row9, best_ref[0:1, 0:_K])
            best_ref[1:2, 0:1] = jnp.where(take, tmax, prev)

        @pl.when(q == nQ - 1)
        def _():
            s = jnp.sqrt(jnp.maximum(best_ref[0:1, 0:_K], 1e-12))
            e = jnp.exp(s - s[:, _K - 1:_K])
            w = 1.0 - e[:, 0:1] / jnp.sum(e, axis=1, keepdims=True)
            score_ref[...] = w * s[:, 0:1]


def kernel(embedding, memory_bank):
    Q, D = embedding.shape
    M = memory_bank.shape[0]
    QT = 448 if Q % 448 == 0 else Q
    C = 2048 if M % 2048 == 0 else M
    nQ, nM = Q // QT, M // C

    outv, outi, score = pl.pallas_call(
        functools.partial(_body, C=C, nM=nM, nQ=nQ),
        grid=(nQ, nM),
        in_specs=[
            pl.BlockSpec((QT, D), lambda q, m: (q, 0)),
            pl.BlockSpec((C, D), lambda q, m: (m, 0)),
        ],
        out_specs=[
            pl.BlockSpec((QT, _K), lambda q, m: (q, 0)),
            pl.BlockSpec((QT, _K), lambda q, m: (q, 0)),
            pl.BlockSpec((1, 1), lambda q, m: (0, 0)),
        ],
        out_shape=[
            jax.ShapeDtypeStruct((Q, _K), jnp.float32),
            jax.ShapeDtypeStruct((Q, _K), jnp.int32),
            jax.ShapeDtypeStruct((1, 1), jnp.float32),
        ],
        scratch_shapes=[pltpu.VMEM((8, 128), jnp.float32)],
        compiler_params=pltpu.CompilerParams(
            dimension_semantics=("arbitrary", "arbitrary")),
    )(embedding, memory_bank)
    return outv, outi, score[0, 0]
